# speedup vs baseline: 2.1476x; 2.1476x over previous
import jax
import jax.numpy as jnp
from jax.experimental import pallas as pl
from jax.experimental.pallas import tpu as pltpu

_B, _NC, _NT, _P = 16, 32, 8, 16
_C, _H, _W = 3, 64, 64
_K = _B * _NT
_D = _NC * _P
_HW = _H * _W
_CHW = _C * _HW
_JT = 1024
_NJ = _CHW // _JT
_BG_PEN = 0.1
_EMPTY_W = 0.1


def _loss_kernel(logits_ref, poses_ref, targets_ref, masks_ref, images_ref,
                 w_ref, b_ref, src_ref, tgt_ref, out_ref,
                 g_scr, wm_scr, acc_ref):
    t = pl.program_id(0)

    @pl.when(t == 0)
    def _init():
        src = src_ref[...]
        tgt = tgt_ref[...]
        poses = poses_ref[...]
        poses_rep = jnp.reshape(
            jnp.broadcast_to(poses[:, None, :], (_B, _NT, _D)), (_K, _D))
        caps = jax.lax.broadcasted_iota(jnp.int32, (_K, _D), 1) // _P
        g_scr[...] = jnp.where(caps == src, poses_rep, 0.0)

        kk = jax.lax.broadcasted_iota(jnp.int32, (_K, _K), 0)
        rr = jax.lax.broadcasted_iota(jnp.int32, (_K, _K), 1)
        sel = jnp.where(rr == (kk // _NT) * _NT + tgt, 1.0, 0.0)
        labels = jnp.sum(sel * targets_ref[...], axis=1, keepdims=True)
        present = jnp.where(labels > 0.5, 1.0, 0.0)
        m = jnp.dot(sel, masks_ref[...], preferred_element_type=jnp.float32)
        wm_scr[...] = (_BG_PEN + (1.0 - _BG_PEN) * m) * present
        acc_ref[0, 0] = 0.0

    recon = jnp.dot(g_scr[...], w_ref[...],
                    preferred_element_type=jnp.float32) + b_ref[...]
    imgs = jnp.reshape(
        jnp.broadcast_to(images_ref[...][:, None, :], (_B, _NT, _JT)),
        (_K, _JT))
    diff = recon - imgs
    p0 = pl.multiple_of((t % (_HW // _JT)) * _JT, _JT)
    wslice = wm_scr[:, pl.ds(p0, _JT)]
    acc_ref[0, 0] += jnp.sum(wslice * diff * diff)

    @pl.when(t == _NJ - 1)
    def _fin():
        src = src_ref[...]
        tgt = tgt_ref[...]
        kk = jax.lax.broadcasted_iota(jnp.int32, (_K, _K), 0)
        rr = jax.lax.broadcasted_iota(jnp.int32, (_K, _K), 1)
        sel = jnp.where(rr == (kk // _NT) * _NT + tgt, 1.0, 0.0)
        labels = jnp.sum(sel * targets_ref[...], axis=1, keepdims=True)

        kk2 = jax.lax.broadcasted_iota(jnp.int32, (_K, _D), 0)
        cc2 = jax.lax.broadcasted_iota(jnp.int32, (_K, _D), 1)
        sel2 = jnp.where(cc2 == (kk2 // _NT) * _NC + src, 1.0, 0.0)
        sl = jnp.sum(sel2 * logits_ref[...], axis=1, keepdims=True)

        wc = jnp.where(labels > 0.5, 1.0, _EMPTY_W)
        per = (jnp.maximum(sl, 0.0) - sl * labels
               + jnp.log1p(jnp.exp(-jnp.abs(sl))))
        loss_cls = jnp.sum(wc * per) / (_K * _NC)
        loss_recon = acc_ref[0, 0] / (_CHW * _NC)
        total = loss_cls + loss_recon
        lane = jax.lax.broadcasted_iota(jnp.int32, (1, 128), 1)
        out_ref[...] = jnp.where(lane == 0, total,
                                 jnp.where(lane == 1, loss_cls, loss_recon))


def _run(logits_row, poses_flat, targets_row, masks_flat, images_flat,
         W_dec, b_row, src_col, tgt_col, interpret=False):
    return pl.pallas_call(
        _loss_kernel,
        grid=(_NJ,),
        in_specs=[
            pl.BlockSpec((1, _D), lambda t: (0, 0)),
            pl.BlockSpec((_B, _D), lambda t: (0, 0)),
            pl.BlockSpec((1, _K), lambda t: (0, 0)),
            pl.BlockSpec((_K, _HW), lambda t: (0, 0)),
            pl.BlockSpec((_B, _JT), lambda t: (0, t)),
            pl.BlockSpec((_D, _JT), lambda t: (0, t)),
            pl.BlockSpec((1, _JT), lambda t: (0, t)),
            pl.BlockSpec((_K, 1), lambda t: (0, 0)),
            pl.BlockSpec((_K, 1), lambda t: (0, 0)),
        ],
        out_specs=pl.BlockSpec((1, 128), lambda t: (0, 0)),
        out_shape=jax.ShapeDtypeStruct((1, 128), jnp.float32),
        scratch_shapes=[
            pltpu.VMEM((_K, _D), jnp.float32),
            pltpu.VMEM((_K, _HW), jnp.float32),
            pltpu.SMEM((1, 1), jnp.float32),
        ],
        interpret=interpret,
    )(logits_row, poses_flat, targets_row, masks_flat, images_flat,
      W_dec, b_row, src_col, tgt_col)


@jax.jit
def kernel(attribute_logits, attribute_poses, visual_attributes_targets,
           va_masks, images, W_dec, b_dec, src_idx, tgt_idx):
    logits_row = attribute_logits.reshape(1, _B * _NC)
    poses_flat = attribute_poses.reshape(_B, _D)
    targets_row = visual_attributes_targets.reshape(1, _K)
    masks_flat = va_masks.reshape(_K, _HW)
    images_flat = images.reshape(_B, _CHW)
    b_row = b_dec.reshape(1, _CHW)
    src_col = src_idx.reshape(_K, 1).astype(jnp.int32)
    tgt_col = tgt_idx.reshape(_K, 1).astype(jnp.int32)
    res = _run(logits_row, poses_flat, targets_row, masks_flat, images_flat,
               W_dec, b_row, src_col, tgt_col)
    return res[0, :3]


# speedup vs baseline: 2.3422x; 1.0907x over previous
import jax
import jax.numpy as jnp
from jax.experimental import pallas as pl
from jax.experimental.pallas import tpu as pltpu

_B, _NC, _NT, _P = 16, 32, 8, 16
_C, _H, _W = 3, 64, 64
_K = _B * _NT
_D = _NC * _P
_HW = _H * _W
_CHW = _C * _HW
_JT = 2048
_NJ = _CHW // _JT
_BG_PEN = 0.1
_EMPTY_W = 0.1


def _loss_kernel(logits_ref, poses_ref, targets_ref, masks_ref, images_ref,
                 w_ref, b_ref, src_ref, tgt_ref, out_ref,
                 g_scr, wm_scr, acc_ref):
    t = pl.program_id(0)

    @pl.when(t == 0)
    def _init():
        src = src_ref[...]
        tgt = tgt_ref[...]
        poses = poses_ref[...]
        poses_rep = jnp.reshape(
            jnp.broadcast_to(poses[:, None, :], (_B, _NT, _D)), (_K, _D))
        caps = jax.lax.broadcasted_iota(jnp.int32, (_K, _D), 1) // _P
        g_scr[...] = jnp.where(caps == src, poses_rep, 0.0)

        kk = jax.lax.broadcasted_iota(jnp.int32, (_K, _K), 0)
        rr = jax.lax.broadcasted_iota(jnp.int32, (_K, _K), 1)
        sel = jnp.where(rr == (kk // _NT) * _NT + tgt, 1.0, 0.0)
        labels = jnp.sum(sel * targets_ref[...], axis=1, keepdims=True)
        present = jnp.where(labels > 0.5, 1.0, 0.0)
        m = jnp.dot(sel, masks_ref[...], preferred_element_type=jnp.float32)
        wm_scr[...] = (_BG_PEN + (1.0 - _BG_PEN) * m) * present
        acc_ref[0, 0] = 0.0

    recon = jnp.dot(g_scr[...], w_ref[...],
                    preferred_element_type=jnp.float32) + b_ref[...]
    imgs = jnp.reshape(
        jnp.broadcast_to(images_ref[...][:, None, :], (_B, _NT, _JT)),
        (_K, _JT))
    diff = recon - imgs
    p0 = pl.multiple_of((t % (_HW // _JT)) * _JT, _JT)
    wslice = wm_scr[:, pl.ds(p0, _JT)]
    acc_ref[0, 0] += jnp.sum(wslice * diff * diff)

    @pl.when(t == _NJ - 1)
    def _fin():
        src = src_ref[...]
        tgt = tgt_ref[...]
        kk = jax.lax.broadcasted_iota(jnp.int32, (_K, _K), 0)
        rr = jax.lax.broadcasted_iota(jnp.int32, (_K, _K), 1)
        sel = jnp.where(rr == (kk // _NT) * _NT + tgt, 1.0, 0.0)
        labels = jnp.sum(sel * targets_ref[...], axis=1, keepdims=True)

        kk2 = jax.lax.broadcasted_iota(jnp.int32, (_K, _D), 0)
        cc2 = jax.lax.broadcasted_iota(jnp.int32, (_K, _D), 1)
        sel2 = jnp.where(cc2 == (kk2 // _NT) * _NC + src, 1.0, 0.0)
        sl = jnp.sum(sel2 * logits_ref[...], axis=1, keepdims=True)

        wc = jnp.where(labels > 0.5, 1.0, _EMPTY_W)
        per = (jnp.maximum(sl, 0.0) - sl * labels
               + jnp.log1p(jnp.exp(-jnp.abs(sl))))
        loss_cls = jnp.sum(wc * per) / (_K * _NC)
        loss_recon = acc_ref[0, 0] / (_CHW * _NC)
        total = loss_cls + loss_recon
        lane = jax.lax.broadcasted_iota(jnp.int32, (1, 128), 1)
        out_ref[...] = jnp.where(lane == 0, total,
                                 jnp.where(lane == 1, loss_cls, loss_recon))


def _run(logits_row, poses_flat, targets_row, masks_flat, images_flat,
         W_dec, b_row, src_col, tgt_col, interpret=False):
    return pl.pallas_call(
        _loss_kernel,
        grid=(_NJ,),
        in_specs=[
            pl.BlockSpec((1, _D), lambda t: (0, 0)),
            pl.BlockSpec((_B, _D), lambda t: (0, 0)),
            pl.BlockSpec((1, _K), lambda t: (0, 0)),
            pl.BlockSpec((_K, _HW), lambda t: (0, 0)),
            pl.BlockSpec((_B, _JT), lambda t: (0, t)),
            pl.BlockSpec((_D, _JT), lambda t: (0, t)),
            pl.BlockSpec((1, _JT), lambda t: (0, t)),
            pl.BlockSpec((_K, 1), lambda t: (0, 0)),
            pl.BlockSpec((_K, 1), lambda t: (0, 0)),
        ],
        out_specs=pl.BlockSpec((1, 128), lambda t: (0, 0)),
        out_shape=jax.ShapeDtypeStruct((1, 128), jnp.float32),
        scratch_shapes=[
            pltpu.VMEM((_K, _D), jnp.float32),
            pltpu.VMEM((_K, _HW), jnp.float32),
            pltpu.SMEM((1, 1), jnp.float32),
        ],
        interpret=interpret,
    )(logits_row, poses_flat, targets_row, masks_flat, images_flat,
      W_dec, b_row, src_col, tgt_col)


@jax.jit
def kernel(attribute_logits, attribute_poses, visual_attributes_targets,
           va_masks, images, W_dec, b_dec, src_idx, tgt_idx):
    logits_row = attribute_logits.reshape(1, _B * _NC)
    poses_flat = attribute_poses.reshape(_B, _D)
    targets_row = visual_attributes_targets.reshape(1, _K)
    masks_flat = va_masks.reshape(_K, _HW)
    images_flat = images.reshape(_B, _CHW)
    b_row = b_dec.reshape(1, _CHW)
    src_col = src_idx.reshape(_K, 1).astype(jnp.int32)
    tgt_col = tgt_idx.reshape(_K, 1).astype(jnp.int32)
    res = _run(logits_row, poses_flat, targets_row, masks_flat, images_flat,
               W_dec, b_row, src_col, tgt_col)
    return res[0, :3]
